# Initial kernel scaffold; baseline (speedup 1.0000x reference)
import functools

import jax
import jax.numpy as jnp
from jax import lax
from jax.experimental import pallas as pl
from jax.experimental.pallas import tpu as pltpu
from jax.experimental.pallas import tpu_sc as plsc

B, L, NVARS = 4096, 200, 2
VOCAB, DIM = 1000000, 32
N = B * L

_info = plsc.get_sparse_core_info()
NC, NS = _info.num_cores, _info.num_subcores
NW = NC * NS
ROWS_PER_W = N // NW
CHUNK = 1024
STEPS = ROWS_PER_W // CHUNK

_mesh = plsc.VectorSubcoreMesh(core_axis_name="c", subcore_axis_name="s")


@functools.partial(
    pl.kernel,
    out_type=jax.ShapeDtypeStruct((N, 2, DIM), jnp.float32),
    mesh=_mesh,
    scratch_types=[
        pltpu.VMEM((CHUNK,), jnp.int32),
        pltpu.VMEM((CHUNK,), jnp.int32),
        pltpu.VMEM((CHUNK, DIM), jnp.float32),
        pltpu.VMEM((CHUNK, DIM), jnp.float32),
        pltpu.SemaphoreType.DMA,
    ],
)
def _embed_sc(idx0_hbm, idx1_hbm, w0_hbm, w1_hbm, out_hbm,
              idx0_v, idx1_v, rows0_v, rows1_v, sem):
    wid = lax.axis_index("s") * NC + lax.axis_index("c")
    w_base = wid * ROWS_PER_W

    def step(c, carry):
        base = w_base + c * CHUNK
        pltpu.sync_copy(idx0_hbm.at[pl.ds(base, CHUNK)], idx0_v)
        pltpu.sync_copy(idx1_hbm.at[pl.ds(base, CHUNK)], idx1_v)
        cp0 = pltpu.async_copy(w0_hbm.at[idx0_v], rows0_v, sem)
        cp1 = pltpu.async_copy(w1_hbm.at[idx1_v], rows1_v, sem)
        cp0.wait()
        cp1.wait()
        pltpu.sync_copy(rows0_v, out_hbm.at[pl.ds(base, CHUNK), 0])
        pltpu.sync_copy(rows1_v, out_hbm.at[pl.ds(base, CHUNK), 1])
        return carry

    lax.fori_loop(0, STEPS, step, 0)


def kernel(X, W0, W1):
    idx = X.reshape(N, NVARS).astype(jnp.int32)
    out = _embed_sc(idx[:, 0], idx[:, 1], W0, W1)
    return out.reshape(B, L, 2 * DIM)


# speedup vs baseline: 1.8172x; 1.8172x over previous
import functools

import jax
import jax.numpy as jnp
from jax import lax
from jax.experimental import pallas as pl
from jax.experimental.pallas import tpu as pltpu
from jax.experimental.pallas import tpu_sc as plsc

B, L, NVARS = 4096, 200, 2
VOCAB, DIM = 1000000, 32
N = B * L

_info = plsc.get_sparse_core_info()
NC, NS = _info.num_cores, _info.num_subcores
NW = NC * NS
ROWS_PER_W = N // NW
CHUNK = 1024
STEPS = ROWS_PER_W // CHUNK

_mesh = plsc.VectorSubcoreMesh(core_axis_name="c", subcore_axis_name="s")


@functools.partial(
    pl.kernel,
    out_type=jax.ShapeDtypeStruct((N, 2, DIM), jnp.float32),
    mesh=_mesh,
    scratch_types=[
        pltpu.VMEM((CHUNK,), jnp.int32),
        pltpu.VMEM((CHUNK,), jnp.int32),
        pltpu.VMEM((CHUNK, DIM), jnp.float32),
        pltpu.VMEM((CHUNK, DIM), jnp.float32),
        pltpu.SemaphoreType.DMA,
    ],
    compiler_params=pltpu.CompilerParams(use_tc_tiling_on_sc=False),
)
def _embed_sc(idx0_hbm, idx1_hbm, w0_hbm, w1_hbm, out_hbm,
              idx0_v, idx1_v, rows0_v, rows1_v, sem):
    wid = lax.axis_index("s") * NC + lax.axis_index("c")
    w_base = wid * ROWS_PER_W

    def step(c, carry):
        base = w_base + c * CHUNK
        pltpu.sync_copy(idx0_hbm.at[pl.ds(base, CHUNK)], idx0_v)
        pltpu.sync_copy(idx1_hbm.at[pl.ds(base, CHUNK)], idx1_v)
        cp0 = pltpu.async_copy(w0_hbm.at[idx0_v], rows0_v, sem)
        cp1 = pltpu.async_copy(w1_hbm.at[idx1_v], rows1_v, sem)
        cp0.wait()
        cp1.wait()
        pltpu.sync_copy(rows0_v, out_hbm.at[pl.ds(base, CHUNK), 0])
        pltpu.sync_copy(rows1_v, out_hbm.at[pl.ds(base, CHUNK), 1])
        return carry

    lax.fori_loop(0, STEPS, step, 0)


def kernel(X, W0, W1):
    idx = X.reshape(N, NVARS).astype(jnp.int32)
    out = _embed_sc(idx[:, 0], idx[:, 1], W0, W1)
    return out.reshape(B, L, 2 * DIM)


# speedup vs baseline: 1.8564x; 1.0216x over previous
import functools

import jax
import jax.numpy as jnp
from jax import lax
from jax.experimental import pallas as pl
from jax.experimental.pallas import tpu as pltpu
from jax.experimental.pallas import tpu_sc as plsc

B, L, NVARS = 4096, 200, 2
VOCAB, DIM = 1000000, 32
N = B * L

_info = plsc.get_sparse_core_info()
NC, NS = _info.num_cores, _info.num_subcores
NW = NC * NS
ROWS_PER_W = N // NW
CHUNK = 800
S = ROWS_PER_W // CHUNK
G = S // 2

_mesh = plsc.VectorSubcoreMesh(core_axis_name="c", subcore_axis_name="s")


@functools.partial(
    pl.kernel,
    out_type=jax.ShapeDtypeStruct((N, 2, DIM), jnp.float32),
    mesh=_mesh,
    scratch_types=[
        pltpu.VMEM((2, CHUNK), jnp.int32),
        pltpu.VMEM((2, CHUNK), jnp.int32),
        pltpu.VMEM((2, CHUNK, DIM), jnp.float32),
        pltpu.VMEM((2, CHUNK, DIM), jnp.float32),
        pltpu.SemaphoreType.DMA,
        pltpu.SemaphoreType.DMA,
        pltpu.SemaphoreType.DMA,
        pltpu.SemaphoreType.DMA,
    ],
    compiler_params=pltpu.CompilerParams(use_tc_tiling_on_sc=False),
)
def _embed_sc(idx0_hbm, idx1_hbm, w0_hbm, w1_hbm, out_hbm,
              idx0_v, idx1_v, rows0_v, rows1_v, sg0, sg1, so0, so1):
    wid = lax.axis_index("s") * NC + lax.axis_index("c")
    w_base = wid * ROWS_PER_W
    sg = (sg0, sg1)
    so = (so0, so1)

    def idx_load(base, s):
        pltpu.sync_copy(idx0_hbm.at[pl.ds(base, CHUNK)], idx0_v.at[s])
        pltpu.sync_copy(idx1_hbm.at[pl.ds(base, CHUNK)], idx1_v.at[s])

    def gather_start(s):
        pltpu.async_copy(w0_hbm.at[idx0_v.at[s]], rows0_v.at[s], sg[s])
        pltpu.async_copy(w1_hbm.at[idx1_v.at[s]], rows1_v.at[s], sg[s])

    def gather_wait(s):
        pltpu.make_async_copy(w0_hbm.at[idx0_v.at[s]], rows0_v.at[s], sg[s]).wait()
        pltpu.make_async_copy(w1_hbm.at[idx1_v.at[s]], rows1_v.at[s], sg[s]).wait()

    def out_start(base, s):
        pltpu.async_copy(rows0_v.at[s], out_hbm.at[pl.ds(base, CHUNK), 0], so[s])
        pltpu.async_copy(rows1_v.at[s], out_hbm.at[pl.ds(base, CHUNK), 1], so[s])

    def out_wait(base, s):
        pltpu.make_async_copy(rows0_v.at[s], out_hbm.at[pl.ds(base, CHUNK), 0], so[s]).wait()
        pltpu.make_async_copy(rows1_v.at[s], out_hbm.at[pl.ds(base, CHUNK), 1], so[s]).wait()

    idx_load(w_base, 0)
    gather_start(0)

    def outer(g, carry):
        b0 = w_base + (2 * g) * CHUNK
        b1 = b0 + CHUNK

        idx_load(b1, 1)

        @pl.when(g > 0)
        def _():
            out_wait(b1 - 2 * CHUNK, 1)

        gather_start(1)
        gather_wait(0)
        out_start(b0, 0)

        @pl.when(g < G - 1)
        def _():
            idx_load(b1 + CHUNK, 0)
            out_wait(b0, 0)
            gather_start(0)

        gather_wait(1)
        out_start(b1, 1)
        return carry

    lax.fori_loop(0, G, outer, 0)

    out_wait(w_base + (S - 2) * CHUNK, 0)
    out_wait(w_base + (S - 1) * CHUNK, 1)


def kernel(X, W0, W1):
    idx = X.reshape(N, NVARS).astype(jnp.int32)
    out = _embed_sc(idx[:, 0], idx[:, 1], W0, W1)
    return out.reshape(B, L, 2 * DIM)
